# Initial kernel scaffold; baseline (speedup 1.0000x reference)
import functools

import jax
import jax.numpy as jnp
from jax import lax
from jax.experimental import pallas as pl
from jax.experimental.pallas import tpu as pltpu
from jax.experimental.pallas import tpu_sc as plsc

NNODE = 1359
NPAD = 1408
EPAD = 16 * 11 * 128
MSP = 16 * 85 * 1408
MOUT = 32 * 59808

_INTERPRET = False


def _mbuild_body(r_hbm, c_hbm, a_hbm, out_hbm,
                 m_s, deg_s, rv, cv, av, wv, valv, idxv, degv, disv, zb):
    cid = lax.axis_index("c")
    sid = lax.axis_index("s")
    wid = sid * 2 + cid

    zeros = jnp.zeros((16,), jnp.float32)
    for i in range(88):
        zb[pl.ds(i * 16, 16)] = zeros
    for i in range(85):
        pltpu.sync_copy(zb, m_s.at[pl.ds((sid * 85 + i) * 1408, 1408)])

    @pl.when(sid == 0)
    def _():
        pltpu.sync_copy(zb, deg_s)

    plsc.subcore_barrier()

    pltpu.sync_copy(r_hbm.at[sid], rv)
    pltpu.sync_copy(c_hbm.at[sid], cv)
    pltpu.sync_copy(a_hbm.at[sid], av)

    for j in range(11):
        for l in range(8):
            sl = pl.ds(l * 16, 16)
            r = rv[j, sl]
            c = cv[j, sl]
            a = av[j, sl]
            wv[j, sl] = jnp.where(r == c, 0.0, a)
    for j in range(11):
        pltpu.sync_copy(wv.at[j], deg_s.at[rv.at[j]], add=True)

    plsc.subcore_barrier()

    pltpu.sync_copy(deg_s, degv)
    for i in range(88):
        sl = pl.ds(i * 16, 16)
        d = degv[sl]
        xi = plsc.bitcast(d, jnp.int32)
        yi = jnp.int32(0x5F3759DF) - (xi >> 1)
        y = plsc.bitcast(yi, jnp.float32)
        y = y * (1.5 - 0.5 * d * y * y)
        y = y * (1.5 - 0.5 * d * y * y)
        y = y * (1.5 - 0.5 * d * y * y)
        disv[sl] = jnp.where(d > 0.0, y, 0.0)

    for j in range(11):
        for l in range(8):
            sl = pl.ds(l * 16, 16)
            r = rv[j, sl]
            c = cv[j, sl]
            w = wv[j, sl]
            dr = plsc.load_gather(disv, [r])
            dc = plsc.load_gather(disv, [c])
            valv[j, sl] = -(dr * w * dc)
            idxv[j, sl] = r * NPAD + c
    for j in range(11):
        pltpu.sync_copy(valv.at[j], m_s.at[idxv.at[j]], add=True)

    plsc.subcore_barrier()

    pltpu.sync_copy(m_s.at[pl.ds(wid * 59808, 59808)],
                    out_hbm.at[pl.ds(wid * 59808, 59808)])


def _build_m(edge_index, edge_attr):
    e = edge_attr.shape[0]
    padn = EPAD - e
    r = jnp.pad(edge_index[0], (0, padn)).reshape(16, 11, 128)
    c = jnp.pad(edge_index[1], (0, padn)).reshape(16, 11, 128)
    a = jnp.pad(edge_attr, (0, padn)).reshape(16, 11, 128)
    mesh = plsc.VectorSubcoreMesh(core_axis_name="c", subcore_axis_name="s")
    fn = pl.kernel(
        _mbuild_body,
        mesh=mesh,
        out_type=jax.ShapeDtypeStruct((MOUT,), jnp.float32),
        scratch_types=[
            pltpu.VMEM_SHARED((MSP,), jnp.float32),
            pltpu.VMEM_SHARED((NPAD,), jnp.float32),
            pltpu.VMEM((11, 128), jnp.int32),
            pltpu.VMEM((11, 128), jnp.int32),
            pltpu.VMEM((11, 128), jnp.float32),
            pltpu.VMEM((11, 128), jnp.float32),
            pltpu.VMEM((11, 128), jnp.float32),
            pltpu.VMEM((11, 128), jnp.int32),
            pltpu.VMEM((NPAD,), jnp.float32),
            pltpu.VMEM((NPAD,), jnp.float32),
            pltpu.VMEM((NPAD,), jnp.float32),
        ],
    )
    mflat = fn(r, c, a)
    m = mflat[: NNODE * NPAD].reshape(NNODE, NPAD)
    return jnp.pad(m, ((0, NPAD - NNODE), (0, 0)))


def _tconv_body(x_ref, w_ref, b_ref, o_ref, *, kt, co):
    t = pl.program_id(0)
    cp = x_ref.shape[1]
    n = x_ref.shape[2]
    xs = x_ref[pl.ds(t, kt)].reshape(kt * cp, n)
    pqr = jnp.dot(w_ref[...], xs, preferred_element_type=jnp.float32)
    pqr = pqr + b_ref[...]
    p = pqr[:co]
    q = pqr[co:2 * co]
    r = pqr[2 * co:]
    o_ref[0] = jax.nn.relu(p * jax.nn.sigmoid(q) + r)


def _tconv(h, p, t_grid):
    co, cin, _, kt = p['w1'].shape
    tin, cp, n = h.shape
    ws = []
    for k in ('w1', 'w2', 'w3'):
        w = jnp.transpose(p[k][:, :, 0, :], (0, 2, 1))
        w = jnp.pad(w, ((0, 0), (0, 0), (0, cp - cin)))
        ws.append(w.reshape(co, kt * cp))
    wt = jnp.concatenate(ws, 0)
    bt = jnp.concatenate([p['b1'], p['b2'], p['b3']])[:, None]
    if t_grid is None:
        t_grid = tin - kt + 1
    return pl.pallas_call(
        functools.partial(_tconv_body, kt=kt, co=co),
        grid=(t_grid,),
        in_specs=[
            pl.BlockSpec((tin, cp, n), lambda t: (0, 0, 0)),
            pl.BlockSpec((3 * co, kt * cp), lambda t: (0, 0)),
            pl.BlockSpec((3 * co, 1), lambda t: (0, 0)),
        ],
        out_specs=pl.BlockSpec((1, co, n), lambda t: (t, 0, 0)),
        out_shape=jax.ShapeDtypeStruct((t_grid, co, n), jnp.float32),
        interpret=_INTERPRET,
    )(h, wt, bt)


def _cheb_body(z_ref, m_ref, w_ref, b_ref, o_ref, a_ref, b2_ref, *, kk, tt, c):
    n = m_ref.shape[0]
    f32 = jnp.float32
    for t in range(tt):
        o_ref[t] = jnp.dot(w_ref[0], z_ref[t], preferred_element_type=f32)
    z2 = z_ref[...].reshape(tt * c, n)
    if kk > 1:
        a_ref[...] = jnp.dot(z2, m_ref[...], preferred_element_type=f32)
        for t in range(tt):
            o_ref[t] += jnp.dot(w_ref[1], a_ref[pl.ds(t * c, c)],
                                preferred_element_type=f32)
    for k in range(2, kk):
        src = a_ref if k % 2 == 0 else b2_ref
        dst = b2_ref if k % 2 == 0 else a_ref
        prev2 = z2 if k == 2 else dst[...]
        dst[...] = 2.0 * jnp.dot(src[...], m_ref[...],
                                 preferred_element_type=f32) - prev2
        for t in range(tt):
            o_ref[t] += jnp.dot(w_ref[k], dst[pl.ds(t * c, c)],
                                preferred_element_type=f32)
    o_ref[...] = jax.nn.relu(o_ref[...] + b_ref[...])


def _cheb(h, m, p, tt):
    tin, c, n = h.shape
    kk, _, f = p['weight'].shape
    wt = jnp.transpose(p['weight'], (0, 2, 1))
    bt = p['bias'][None, :, None]
    grid = tin // tt
    return pl.pallas_call(
        functools.partial(_cheb_body, kk=kk, tt=tt, c=c),
        grid=(grid,),
        in_specs=[
            pl.BlockSpec((tt, c, n), lambda i: (i, 0, 0)),
            pl.BlockSpec((n, n), lambda i: (0, 0)),
            pl.BlockSpec((kk, f, c), lambda i: (0, 0, 0)),
            pl.BlockSpec((1, f, 1), lambda i: (0, 0, 0)),
        ],
        out_specs=pl.BlockSpec((tt, f, n), lambda i: (i, 0, 0)),
        out_shape=jax.ShapeDtypeStruct((tin, f, n), jnp.float32),
        scratch_shapes=[
            pltpu.VMEM((tt * c, n), jnp.float32),
            pltpu.VMEM((tt * c, n), jnp.float32),
        ],
        interpret=_INTERPRET,
    )(h, m, wt, bt)


def _bn_body(x_ref, g_ref, b_ref, o_ref, *, eps):
    tt, c, n = x_ref.shape
    x2 = x_ref[...].reshape(tt * c, n)
    mu = jnp.mean(x2, axis=0, keepdims=True)
    var = jnp.maximum(jnp.mean(x2 * x2, axis=0, keepdims=True) - mu * mu, 0.0)
    y = (x2 - mu) * lax.rsqrt(var + eps) * g_ref[...] + b_ref[...]
    o_ref[...] = y.reshape(tt, c, n)


def _bn(h, g, b):
    tt, c, n = h.shape
    gp = jnp.pad(g, (0, n - g.shape[0]))[None, :]
    bp = jnp.pad(b, (0, n - b.shape[0]))[None, :]
    return pl.pallas_call(
        functools.partial(_bn_body, eps=1e-5),
        out_shape=jax.ShapeDtypeStruct((tt, c, n), jnp.float32),
        interpret=_INTERPRET,
    )(h, gp, bp)


def _fc_body(x_ref, w_ref, b_ref, o_ref):
    for t in range(x_ref.shape[0]):
        o_ref[t] = jnp.dot(w_ref[...], x_ref[t],
                           preferred_element_type=jnp.float32) + b_ref[...]


def _fc(h, w, b):
    tt, c, n = h.shape
    fo = w.shape[0]
    fp = 8
    wp = jnp.pad(w, ((0, fp - fo), (0, 0)))
    bp = jnp.pad(b, (0, fp - fo))[:, None]
    return pl.pallas_call(
        _fc_body,
        out_shape=jax.ShapeDtypeStruct((tt, fp, n), jnp.float32),
        interpret=_INTERPRET,
    )(h, wp, bp)


def _st_block(h, m, p, t1_grid, tt_cheb):
    h = _tconv(h, p['t1'], t1_grid)
    h = _cheb(h, m, p['cheb'], tt_cheb)
    h = _tconv(h, p['t2'], None)
    h = _bn(h, p['bn_g'], p['bn_b'])
    return h


def _forward_dense(x, m, params):
    xt = jnp.transpose(x[0], (0, 2, 1))
    h = jnp.pad(xt, ((0, 0), (0, 2), (0, NPAD - NNODE)))
    h = _st_block(h, m, params['b1'], 64, 8)
    h = _st_block(h, m, params['b2'], None, 4)
    h = _st_block(h, m, params['b3'], None, 8)
    y = _fc(h, params['fc_w'], params['fc_b'])
    y = y[:, :3, :NNODE]
    return jnp.transpose(y, (0, 2, 1))[None]


def kernel(x, edge_index, edge_attr, params):
    m = _build_m(edge_index, edge_attr)
    return _forward_dense(x, m, params)


# speedup vs baseline: 18.9837x; 18.9837x over previous
import functools

import jax
import jax.numpy as jnp
from jax import lax
from jax.experimental import pallas as pl
from jax.experimental.pallas import tpu as pltpu
from jax.experimental.pallas import tpu_sc as plsc

NNODE = 1359
NPAD = 1408
EPAD = 16 * 11 * 128
MW = 1360
HREAL = 680
MSPH = 16 * 43 * MW
HOUT = HREAL * MW
MOUT = 2 * HOUT

_INTERPRET = False


def _mbuild_body(r_hbm, c_hbm, a_hbm, out_hbm,
                 m_s, deg_s, dis_s, rv, cv, av, wv, valv, idxv,
                 degv, disv, drv, dcv, zb, outv):
    cid = lax.axis_index("c")
    sid = lax.axis_index("s")

    zeros = jnp.zeros((16,), jnp.float32)
    for i in range(85):
        zb[pl.ds(i * 16, 16)] = zeros
    for i in range(43):
        pltpu.sync_copy(zb, m_s.at[pl.ds((sid * 43 + i) * MW, MW)])

    @pl.when(sid == 0)
    def _():
        pltpu.sync_copy(zb, deg_s.at[pl.ds(0, MW)])
        pltpu.sync_copy(zb.at[pl.ds(0, 176)], deg_s.at[pl.ds(MW, 176)])

    plsc.subcore_barrier()

    pltpu.sync_copy(r_hbm.at[sid], rv)
    pltpu.sync_copy(c_hbm.at[sid], cv)
    pltpu.sync_copy(a_hbm.at[sid], av)

    for j in range(11):
        for l in range(8):
            sl = pl.ds(l * 16, 16)
            r = rv[j, sl]
            c = cv[j, sl]
            a = av[j, sl]
            wv[j, sl] = jnp.where(r == c, 0.0, a)
    for j in range(11):
        pltpu.sync_copy(wv.at[j], deg_s.at[rv.at[j]], add=True)

    plsc.subcore_barrier()

    pltpu.sync_copy(deg_s.at[pl.ds(sid * 96, 96)], degv)
    for i in range(6):
        sl = pl.ds(i * 16, 16)
        d = degv[sl]
        xi = lax.bitcast_convert_type(d, jnp.int32)
        yi = jnp.int32(0x5F3759DF) - (xi >> 1)
        y = lax.bitcast_convert_type(yi, jnp.float32)
        y = y * (1.5 - 0.5 * d * y * y)
        y = y * (1.5 - 0.5 * d * y * y)
        y = y * (1.5 - 0.5 * d * y * y)
        disv[sl] = jnp.where(d > 0.0, y, 0.0)
    pltpu.sync_copy(disv, dis_s.at[pl.ds(sid * 96, 96)])

    plsc.subcore_barrier()

    base = jnp.full((16,), cid * HREAL, jnp.int32)
    dump = jnp.full((16,), HREAL * MW, jnp.int32)
    for j in range(11):
        pltpu.sync_copy(dis_s.at[rv.at[j]], drv.at[j])
        pltpu.sync_copy(dis_s.at[cv.at[j]], dcv.at[j])
    for j in range(11):
        for l in range(8):
            sl = pl.ds(l * 16, 16)
            r = rv[j, sl]
            c = cv[j, sl]
            w = wv[j, sl]
            dr = drv[j, sl]
            dc = dcv[j, sl]
            valv[j, sl] = -(dr * w * dc)
            rb = r - base
            inh = (rb >= 0) & (rb < HREAL)
            idxv[j, sl] = jnp.where(inh, rb * MW + c, dump)
    for j in range(11):
        pltpu.sync_copy(valv.at[j], m_s.at[idxv.at[j]], add=True)

    plsc.subcore_barrier()

    src0 = sid * 57800
    dst0 = (cid * 16 + sid) * 57800
    for i in range(7):
        pltpu.sync_copy(m_s.at[pl.ds(src0 + i * 8192, 8192)], outv)
        pltpu.sync_copy(outv, out_hbm.at[pl.ds(dst0 + i * 8192, 8192)])
    rem = 57800 - 7 * 8192
    pltpu.sync_copy(m_s.at[pl.ds(src0 + 7 * 8192, rem)],
                    outv.at[pl.ds(0, rem)])
    pltpu.sync_copy(outv.at[pl.ds(0, rem)],
                    out_hbm.at[pl.ds(dst0 + 7 * 8192, rem)])


def _build_m(edge_index, edge_attr):
    e = edge_attr.shape[0]
    padn = EPAD - e
    r = jnp.pad(edge_index[0], (0, padn)).reshape(16, 11, 128)
    c = jnp.pad(edge_index[1], (0, padn)).reshape(16, 11, 128)
    a = jnp.pad(edge_attr, (0, padn)).reshape(16, 11, 128)
    mesh = plsc.VectorSubcoreMesh(core_axis_name="c", subcore_axis_name="s")
    fn = pl.kernel(
        _mbuild_body,
        mesh=mesh,
        out_type=jax.ShapeDtypeStruct((MOUT,), jnp.float32),
        scratch_types=[
            pltpu.VMEM_SHARED((MSPH,), jnp.float32),
            pltpu.VMEM_SHARED((1536,), jnp.float32),
            pltpu.VMEM_SHARED((1536,), jnp.float32),
            pltpu.VMEM((11, 128), jnp.int32),
            pltpu.VMEM((11, 128), jnp.int32),
            pltpu.VMEM((11, 128), jnp.float32),
            pltpu.VMEM((11, 128), jnp.float32),
            pltpu.VMEM((11, 128), jnp.float32),
            pltpu.VMEM((11, 128), jnp.int32),
            pltpu.VMEM((96,), jnp.float32),
            pltpu.VMEM((96,), jnp.float32),
            pltpu.VMEM((11, 128), jnp.float32),
            pltpu.VMEM((11, 128), jnp.float32),
            pltpu.VMEM((1360,), jnp.float32),
            pltpu.VMEM((8192,), jnp.float32),
        ],
    )
    mflat = fn(r, c, a)
    m = mflat.reshape(2 * HREAL, MW)[:NNODE]
    return jnp.pad(m, ((0, NPAD - NNODE), (0, NPAD - MW)))


def _tconv_body(x_ref, w_ref, b_ref, o_ref, *, kt, co):
    t = pl.program_id(0)
    cp = x_ref.shape[1]
    n = x_ref.shape[2]
    xs = x_ref[pl.ds(t, kt)].reshape(kt * cp, n)
    pqr = jnp.dot(w_ref[...], xs, preferred_element_type=jnp.float32)
    pqr = pqr + b_ref[...]
    p = pqr[:co]
    q = pqr[co:2 * co]
    r = pqr[2 * co:]
    o_ref[0] = jax.nn.relu(p * jax.nn.sigmoid(q) + r)


def _tconv(h, p, t_grid):
    co, cin, _, kt = p['w1'].shape
    tin, cp, n = h.shape
    ws = []
    for k in ('w1', 'w2', 'w3'):
        w = jnp.transpose(p[k][:, :, 0, :], (0, 2, 1))
        w = jnp.pad(w, ((0, 0), (0, 0), (0, cp - cin)))
        ws.append(w.reshape(co, kt * cp))
    wt = jnp.concatenate(ws, 0)
    bt = jnp.concatenate([p['b1'], p['b2'], p['b3']])[:, None]
    if t_grid is None:
        t_grid = tin - kt + 1
    return pl.pallas_call(
        functools.partial(_tconv_body, kt=kt, co=co),
        grid=(t_grid,),
        in_specs=[
            pl.BlockSpec((tin, cp, n), lambda t: (0, 0, 0)),
            pl.BlockSpec((3 * co, kt * cp), lambda t: (0, 0)),
            pl.BlockSpec((3 * co, 1), lambda t: (0, 0)),
        ],
        out_specs=pl.BlockSpec((1, co, n), lambda t: (t, 0, 0)),
        out_shape=jax.ShapeDtypeStruct((t_grid, co, n), jnp.float32),
        interpret=_INTERPRET,
    )(h, wt, bt)


def _doth(a, b):
    return jnp.dot(a, b, preferred_element_type=jnp.float32,
                   precision=lax.Precision.HIGHEST)


def _dotd(a, b):
    return jnp.dot(a, b, preferred_element_type=jnp.float32)


def _cheb_body(z_ref, m_ref, w_ref, b_ref, o_ref, a_ref, b2_ref, *, kk, tt, c):
    for t in range(tt):
        o_ref[t] = _dotd(w_ref[0], z_ref[t])
    if kk > 1:
        for t in range(tt):
            sl = pl.ds(t * c, c)
            a_ref[sl] = _doth(z_ref[t], m_ref[...])
            o_ref[t] += _dotd(w_ref[1], a_ref[sl])
    if kk > 2:
        for t in range(tt):
            b2_ref[pl.ds(t * c, c)] = z_ref[t]

        def hop(k, carry):
            for t in range(tt):
                sl = pl.ds(t * c, c)
                new = 2.0 * _doth(a_ref[sl], m_ref[...]) - b2_ref[sl]
                b2_ref[sl] = a_ref[sl]
                a_ref[sl] = new
                o_ref[t] += _dotd(w_ref[k], new)
            return carry

        lax.fori_loop(2, kk, hop, 0)
    o_ref[...] = jax.nn.relu(o_ref[...] + b_ref[...])


def _cheb(h, m, p, tt):
    tin, c, n = h.shape
    kk, _, f = p['weight'].shape
    wt = jnp.transpose(p['weight'], (0, 2, 1))
    bt = p['bias'][None, :, None]
    grid = tin // tt
    return pl.pallas_call(
        functools.partial(_cheb_body, kk=kk, tt=tt, c=c),
        grid=(grid,),
        in_specs=[
            pl.BlockSpec((tt, c, n), lambda i: (i, 0, 0)),
            pl.BlockSpec((n, n), lambda i: (0, 0)),
            pl.BlockSpec((kk, f, c), lambda i: (0, 0, 0)),
            pl.BlockSpec((1, f, 1), lambda i: (0, 0, 0)),
        ],
        out_specs=pl.BlockSpec((tt, f, n), lambda i: (i, 0, 0)),
        out_shape=jax.ShapeDtypeStruct((tin, f, n), jnp.float32),
        scratch_shapes=[
            pltpu.VMEM((tt * c, n), jnp.float32),
            pltpu.VMEM((tt * c, n), jnp.float32),
        ],
        interpret=_INTERPRET,
    )(h, m, wt, bt)


def _bn_body(x_ref, g_ref, b_ref, o_ref, *, eps):
    tt, c, n = x_ref.shape
    x2 = x_ref[...].reshape(tt * c, n)
    mu = jnp.mean(x2, axis=0, keepdims=True)
    var = jnp.maximum(jnp.mean(x2 * x2, axis=0, keepdims=True) - mu * mu, 0.0)
    y = (x2 - mu) * lax.rsqrt(var + eps) * g_ref[...] + b_ref[...]
    o_ref[...] = y.reshape(tt, c, n)


def _bn(h, g, b):
    tt, c, n = h.shape
    gp = jnp.pad(g, (0, n - g.shape[0]))[None, :]
    bp = jnp.pad(b, (0, n - b.shape[0]))[None, :]
    return pl.pallas_call(
        functools.partial(_bn_body, eps=1e-5),
        out_shape=jax.ShapeDtypeStruct((tt, c, n), jnp.float32),
        interpret=_INTERPRET,
    )(h, gp, bp)


def _fc_body(x_ref, w_ref, b_ref, o_ref):
    for t in range(x_ref.shape[0]):
        o_ref[t] = jnp.dot(w_ref[...], x_ref[t],
                           preferred_element_type=jnp.float32) + b_ref[...]


def _fc(h, w, b):
    tt, c, n = h.shape
    fo = w.shape[0]
    fp = 8
    wp = jnp.pad(w, ((0, fp - fo), (0, 0)))
    bp = jnp.pad(b, (0, fp - fo))[:, None]
    return pl.pallas_call(
        _fc_body,
        out_shape=jax.ShapeDtypeStruct((tt, fp, n), jnp.float32),
        interpret=_INTERPRET,
    )(h, wp, bp)


def _st_block(h, m, p, t1_grid, tt_cheb, t2_grid):
    h = _tconv(h, p['t1'], t1_grid)
    h = _cheb(h, m, p['cheb'], tt_cheb)
    h = _tconv(h, p['t2'], t2_grid)
    h = _bn(h, p['bn_g'], p['bn_b'])
    return h


def _forward_dense(x, m, params):
    xt = jnp.transpose(x[0], (0, 2, 1))
    h = jnp.pad(xt, ((0, 0), (0, 2), (0, NPAD - NNODE)))
    h = _st_block(h, m, params['b1'], 64, 8, 26)
    h = _st_block(h, m, params['b2'], None, 2, None)
    h = _st_block(h, m, params['b3'], None, 8, None)
    y = _fc(h, params['fc_w'], params['fc_b'])
    y = y[:, :3, :NNODE]
    return jnp.transpose(y, (0, 2, 1))[None]


def kernel(x, edge_index, edge_attr, params):
    m = _build_m(edge_index, edge_attr)
    return _forward_dense(x, m, params)
